# baseline (speedup 1.0000x reference)
import functools

import jax
import jax.numpy as jnp
from jax import lax
from jax.experimental import pallas as pl
from jax.experimental.pallas import tpu as pltpu

_REG_MAX = 16
_TOPK = 13
_FOCAL_ALPHA = 0.25
_MARGIN = 0.2
_NEG_INF = -1e9
_PI = 3.14159265358979323846


def _sig(x):
    return 1.0 / (1.0 + jnp.exp(-x))


def _atan_pos(t):
    c1 = t > 2.414213562373095
    c2 = t > 0.4142135623730951
    x = jnp.where(c1, -1.0 / t, jnp.where(c2, (t - 1.0) / (t + 1.0), t))
    y0 = jnp.where(c1, _PI / 2, jnp.where(c2, _PI / 4, 0.0))
    z = x * x
    y = ((8.05374449538e-2 * z - 1.38776856032e-1) * z + 1.99777106478e-1) * z - 3.33329491539e-1
    return y0 + x + x * z * y


def _loss_body(gt_ref, gtbT_ref, ohT_ref, valid_ref, vcmc_ref, sT_ref,
               dT_ref, pkT_ref, out_ref, acc_ref, *, B, P, C, G):
    b = pl.program_id(0)

    @pl.when(b == 0)
    def _init():
        for k in range(8):
            acc_ref[k] = 0.0

    s = jnp.transpose(sT_ref[0])
    d = dT_ref[0]
    pk = pkT_ref[0]
    gt = gt_ref[0]
    gtbT = gtbT_ref[0]
    ohT = ohT_ref[0]
    validc = valid_ref[0]
    vcmc = vcmc_ref[0]

    bx1 = pk[0:1, :]
    by1 = pk[1:2, :]
    bx2 = pk[2:3, :]
    by2 = pk[3:4, :]
    obj = pk[4:5, :]
    ax = pk[5:6, :]
    ay = pk[6:7, :]
    st = pk[7:8, :]
    gx1 = gt[:, 0:1]
    gy1 = gt[:, 1:2]
    gx2 = gt[:, 2:3]
    gy2 = gt[:, 3:4]

    ix1 = jnp.maximum(bx1, gx1)
    iy1 = jnp.maximum(by1, gy1)
    ix2 = jnp.minimum(bx2, gx2)
    iy2 = jnp.minimum(by2, gy2)
    inter = jnp.maximum(ix2 - ix1, 0.0) * jnp.maximum(iy2 - iy1, 0.0)
    pa = jnp.maximum(bx2 - bx1, 0.0) * jnp.maximum(by2 - by1, 0.0)
    ga = jnp.maximum(gx2 - gx1, 0.0) * jnp.maximum(gy2 - gy1, 0.0)
    iou = inter / (pa + ga - inter + 1e-7)
    raw_cls = jnp.dot(ohT, s, preferred_element_type=jnp.float32)
    cls_sig = _sig(raw_cls)
    iou2 = iou * iou
    iou6 = iou2 * iou2 * iou2
    inside = (ax >= gx1) & (ax <= gx2) & (ay >= gy1) & (ay <= gy2)
    align = jnp.where(inside & (validc > 0.0), cls_sig * iou6, _NEG_INF)

    work = align
    for _ in range(_TOPK - 1):
        m = jnp.max(work, axis=1, keepdims=True)
        work = jnp.where(work == m, -2e9, work)
    thr = jnp.max(work, axis=1, keepdims=True)

    sel = (align >= thr) & (align > _NEG_INF / 2)
    sal = jnp.where(sel, align, _NEG_INF)
    rm = jnp.max(sal, axis=0, keepdims=True)
    fg = rm > _NEG_INF / 2
    fgf = fg.astype(jnp.float32)
    gidx = lax.broadcasted_iota(jnp.int32, (G, P), 0).astype(jnp.float32)
    cand = jnp.where((sal == rm) & fg, gidx, 1e9)
    gstar = jnp.min(cand, axis=0, keepdims=True)
    ohg = ((gidx == gstar) & fg).astype(jnp.float32)
    tb = jnp.dot(gtbT, ohg, preferred_element_type=jnp.float32)
    cnt = jnp.sum(fgf)

    prob = _sig(obj)
    ce_o = jnp.maximum(obj, 0.0) - obj * fgf + jnp.log(1.0 + jnp.exp(-jnp.abs(obj)))
    p_t = prob * fgf + (1.0 - prob) * (1.0 - fgf)
    af = _FOCAL_ALPHA * fgf + (1.0 - _FOCAL_ALPHA) * (1.0 - fgf)
    one_mpt = 1.0 - p_t
    obj_loss = jnp.sum(ce_o * af * one_mpt * one_mpt) / float(P)

    ones_c = jnp.full((1, C), 1.0, jnp.float32)
    ones_g = jnp.full((1, G), 1.0, jnp.float32)

    masked = jnp.where(vcmc > 0.0, s, -10000.0)
    m_row = jnp.max(masked, axis=0, keepdims=True)
    ssum = jnp.dot(ones_c, jnp.exp(masked - m_row),
                   preferred_element_type=jnp.float32)
    lse = m_row + jnp.log(ssum)
    s_at = jnp.dot(ones_g, ohg * raw_cls,
                   preferred_element_type=jnp.float32)
    match_sum = jnp.sum((lse - s_at) * fgf)

    sig_at = jnp.dot(ones_g, ohg * cls_sig,
                     preferred_element_type=jnp.float32)
    pos_sum_b = jnp.sum(prob * sig_at * fgf)
    masked2 = jnp.where(vcmc > 0.0, s, -1e30)
    m2 = jnp.max(masked2, axis=0, keepdims=True)
    anyv = jnp.max(vcmc) > 0.0
    ncm = jnp.where(anyv, _sig(m2), 0.0)
    neg_sum_b = jnp.sum(prob * ncm * (1.0 - fgf))

    tx1 = tb[0:1, :]
    ty1 = tb[1:2, :]
    tx2 = tb[2:3, :]
    ty2 = tb[3:4, :]
    jx1 = jnp.maximum(bx1, tx1)
    jy1 = jnp.maximum(by1, ty1)
    jx2 = jnp.minimum(bx2, tx2)
    jy2 = jnp.minimum(by2, ty2)
    jint = jnp.maximum(jx2 - jx1, 0.0) * jnp.maximum(jy2 - jy1, 0.0)
    ta = jnp.maximum(tx2 - tx1, 0.0) * jnp.maximum(ty2 - ty1, 0.0)
    iou_r = jint / (pa + ta - jint + 1e-7)
    cw = jnp.maximum(bx2, tx2) - jnp.minimum(bx1, tx1)
    ch = jnp.maximum(by2, ty2) - jnp.minimum(by1, ty1)
    c2 = cw * cw + ch * ch + 1e-7
    rho2 = ((tx1 + tx2 - bx1 - bx2) ** 2 + (ty1 + ty2 - by1 - by2) ** 2) / 4.0
    w1 = jnp.maximum(bx2 - bx1, 1e-7)
    h1 = jnp.maximum(by2 - by1, 1e-7)
    w2 = jnp.maximum(tx2 - tx1, 1e-7)
    h2 = jnp.maximum(ty2 - ty1, 1e-7)
    dat = _atan_pos(w2 / h2) - _atan_pos(w1 / h1)
    v = (4.0 / (_PI * _PI)) * dat * dat
    alpha = v / (v - iou_r + (1.0 + 1e-7))
    ciou = iou_r - (rho2 / c2 + v * alpha)
    iou_sum = jnp.sum((1.0 - ciou) * fgf)

    x_full = d.astype(jnp.float32)
    dstack = jnp.concatenate([
        jnp.clip((ax - tx1) / st, 0.0, _REG_MAX - 1 - 0.01),
        jnp.clip((ay - ty1) / st, 0.0, _REG_MAX - 1 - 0.01),
        jnp.clip((tx2 - ax) / st, 0.0, _REG_MAX - 1 - 0.01),
        jnp.clip((ty2 - ay) / st, 0.0, _REG_MAX - 1 - 0.01),
    ], axis=0)
    grp = lax.broadcasted_iota(jnp.int32, (4, 4 * _REG_MAX), 1) // _REG_MAX
    smat = (grp == lax.broadcasted_iota(jnp.int32, (4, 4 * _REG_MAX), 0)
            ).astype(jnp.float32)
    dmatg = lax.broadcasted_iota(jnp.int32, (4 * _REG_MAX, 4), 0) // _REG_MAX
    dmat = (dmatg == lax.broadcasted_iota(jnp.int32, (4 * _REG_MAX, 4), 1)
            ).astype(jnp.float32)
    r16 = (lax.broadcasted_iota(jnp.int32, (4 * _REG_MAX, 1), 0)
           % _REG_MAX).astype(jnp.float32)
    mxg = jnp.concatenate(
        [jnp.max(x_full[k * _REG_MAX:(k + 1) * _REG_MAX, :], axis=0,
                 keepdims=True) for k in range(4)], axis=0)
    mxb = jnp.dot(dmat, mxg, preferred_element_type=jnp.float32)
    e = jnp.exp(x_full - mxb)
    sums = jnp.dot(smat, e, preferred_element_type=jnp.float32)
    lseg = mxg + jnp.log(sums)
    dful = jnp.dot(dmat, dstack, preferred_element_type=jnp.float32)
    w = jnp.maximum(1.0 - jnp.abs(r16 - dful), 0.0)
    wxs = jnp.dot(smat, w * x_full, preferred_element_type=jnp.float32)
    dfl_sum = jnp.sum((lseg - wxs) * fgf)

    has = cnt > 0.0
    acc_ref[0] = acc_ref[0] + obj_loss
    acc_ref[1] = acc_ref[1] + jnp.where(has, match_sum / jnp.maximum(cnt, 1.0), 0.0)
    acc_ref[2] = acc_ref[2] + jnp.where(has, iou_sum / jnp.maximum(cnt, 1.0), 0.0)
    acc_ref[3] = acc_ref[3] + jnp.where(has, dfl_sum / jnp.maximum(4.0 * cnt, 1.0), 0.0)
    acc_ref[4] = acc_ref[4] + pos_sum_b
    acc_ref[5] = acc_ref[5] + neg_sum_b
    acc_ref[6] = acc_ref[6] + cnt
    acc_ref[7] = acc_ref[7] + (float(P) - cnt)

    @pl.when(b == B - 1)
    def _fin():
        pos_mean = acc_ref[4] / jnp.maximum(acc_ref[6], 1.0)
        neg_mean = acc_ref[5] / jnp.maximum(acc_ref[7], 1.0)
        contrast = jnp.maximum(neg_mean - pos_mean + _MARGIN, 0.0)
        total = acc_ref[0] + acc_ref[1] + acc_ref[2] + acc_ref[3] + contrast
        lane = lax.broadcasted_iota(jnp.int32, (1, 8), 1)
        outv = (jnp.where(lane == 0, acc_ref[0], 0.0)
                + jnp.where(lane == 1, acc_ref[1], 0.0)
                + jnp.where(lane == 2, acc_ref[2], 0.0)
                + jnp.where(lane == 3, acc_ref[3], 0.0)
                + jnp.where(lane == 4, contrast, 0.0)
                + jnp.where(lane == 5, total, 0.0))
        out_ref[...] = outv


def kernel(pred_boxes, pred_scores, pred_objectness, anchor_points,
           stride_tensor, box_distribution, class_mask, gt_boxes, gt_labels):
    B, P, C = pred_scores.shape
    G = gt_boxes.shape[1]
    f32 = jnp.float32

    cls = gt_labels.astype(jnp.int32)
    cls_c = jnp.clip(cls, 0, C - 1)
    vcmc = class_mask.astype(f32).reshape(B, C, 1)
    valid = ((cls >= 0) & (cls < C)
             & jnp.take_along_axis(class_mask, cls_c, axis=1)).astype(f32)
    valid = valid.reshape(B, G, 1)
    onehotT = (cls_c[:, :, None]
               == jnp.arange(C, dtype=jnp.int32)[None, None, :]).astype(f32)
    gtb = gt_boxes.astype(f32)
    gtbT = jnp.transpose(gtb, (0, 2, 1))
    axr = jnp.broadcast_to(anchor_points[:, 0][None], (B, P))
    ayr = jnp.broadcast_to(anchor_points[:, 1][None], (B, P))
    str_ = jnp.broadcast_to(stride_tensor[None], (B, P))
    packT = jnp.stack(
        [pred_boxes[..., 0], pred_boxes[..., 1], pred_boxes[..., 2],
         pred_boxes[..., 3], pred_objectness, axr, ayr, str_],
        axis=1).astype(f32)
    scoresT = pred_scores.astype(f32)
    distT = jnp.transpose(box_distribution.astype(f32), (0, 2, 1))

    body = functools.partial(_loss_body, B=B, P=P, C=C, G=G)
    out = pl.pallas_call(
        body,
        grid=(B,),
        in_specs=[
            pl.BlockSpec((1, G, 4), lambda b: (b, 0, 0)),
            pl.BlockSpec((1, 4, G), lambda b: (b, 0, 0)),
            pl.BlockSpec((1, G, C), lambda b: (b, 0, 0)),
            pl.BlockSpec((1, G, 1), lambda b: (b, 0, 0)),
            pl.BlockSpec((1, C, 1), lambda b: (b, 0, 0)),
            pl.BlockSpec((1, P, C), lambda b: (b, 0, 0)),
            pl.BlockSpec((1, 4 * _REG_MAX, P), lambda b: (b, 0, 0)),
            pl.BlockSpec((1, 8, P), lambda b: (b, 0, 0)),
        ],
        out_specs=pl.BlockSpec((1, 8), lambda b: (0, 0)),
        out_shape=jax.ShapeDtypeStruct((1, 8), f32),
        scratch_shapes=[pltpu.SMEM((8,), f32)],
    )(gtb, gtbT, onehotT, valid, vcmc, scoresT, distT, packT)
    return out[0, :6]


# speedup vs baseline: 1.5756x; 1.5756x over previous
import functools

import jax
import jax.numpy as jnp
from jax import lax
from jax.experimental import pallas as pl
from jax.experimental.pallas import tpu as pltpu

_REG_MAX = 16
_TOPK = 13
_FOCAL_ALPHA = 0.25
_MARGIN = 0.2
_NEG_INF = -1e9
_PI = 3.14159265358979323846


def _sig(x):
    return 1.0 / (1.0 + jnp.exp(-x))


def _atan_pos(t):
    c1 = t > 2.414213562373095
    c2 = t > 0.4142135623730951
    x = jnp.where(c1, -1.0 / t, jnp.where(c2, (t - 1.0) / (t + 1.0), t))
    y0 = jnp.where(c1, _PI / 2, jnp.where(c2, _PI / 4, 0.0))
    z = x * x
    y = ((8.05374449538e-2 * z - 1.38776856032e-1) * z + 1.99777106478e-1) * z - 3.33329491539e-1
    return y0 + x + x * z * y


def _loss_body(gt_ref, gtbT_ref, ohT_ref, valid_ref, vcmc_ref, sT_ref,
               dT_ref, pkT_ref, out_ref, acc_ref, *, B, P, C, G):
    b = pl.program_id(0)

    @pl.when(b == 0)
    def _init():
        for k in range(8):
            acc_ref[k] = 0.0

    s = sT_ref[0]
    d = dT_ref[0]
    pk = pkT_ref[0]
    gt = gt_ref[0]
    gtbT = gtbT_ref[0]
    ohT = ohT_ref[0]
    validc = valid_ref[0]
    vcmc = vcmc_ref[0]

    bx1 = pk[0:1, :]
    by1 = pk[1:2, :]
    bx2 = pk[2:3, :]
    by2 = pk[3:4, :]
    obj = pk[4:5, :]
    ax = pk[5:6, :]
    ay = pk[6:7, :]
    st = pk[7:8, :]
    gx1 = gt[:, 0:1]
    gy1 = gt[:, 1:2]
    gx2 = gt[:, 2:3]
    gy2 = gt[:, 3:4]

    ix1 = jnp.maximum(bx1, gx1)
    iy1 = jnp.maximum(by1, gy1)
    ix2 = jnp.minimum(bx2, gx2)
    iy2 = jnp.minimum(by2, gy2)
    inter = jnp.maximum(ix2 - ix1, 0.0) * jnp.maximum(iy2 - iy1, 0.0)
    pa = jnp.maximum(bx2 - bx1, 0.0) * jnp.maximum(by2 - by1, 0.0)
    ga = jnp.maximum(gx2 - gx1, 0.0) * jnp.maximum(gy2 - gy1, 0.0)
    iou = inter / (pa + ga - inter + 1e-7)
    raw_cls = jnp.dot(ohT, s, preferred_element_type=jnp.float32)
    cls_sig = _sig(raw_cls)
    iou2 = iou * iou
    iou6 = iou2 * iou2 * iou2
    inside = (ax >= gx1) & (ax <= gx2) & (ay >= gy1) & (ay <= gy2)
    align = jnp.where(inside & (validc > 0.0), cls_sig * iou6, _NEG_INF)

    thr = jnp.max(align, axis=1, keepdims=True)
    for _ in range(_TOPK - 1):
        thr = jnp.max(jnp.where(align < thr, align, -2e9), axis=1,
                      keepdims=True)

    sel = (align >= thr) & (align > _NEG_INF / 2)
    sal = jnp.where(sel, align, _NEG_INF)
    rm = jnp.max(sal, axis=0, keepdims=True)
    fg = rm > _NEG_INF / 2
    fgf = fg.astype(jnp.float32)
    gidx = lax.broadcasted_iota(jnp.int32, (G, P), 0).astype(jnp.float32)
    cand = jnp.where((sal == rm) & fg, gidx, 1e9)
    gstar = jnp.min(cand, axis=0, keepdims=True)
    ohg = ((gidx == gstar) & fg).astype(jnp.float32)
    tb = jnp.dot(gtbT, ohg, preferred_element_type=jnp.float32)
    cnt = jnp.sum(fgf)

    prob = _sig(obj)
    ce_o = jnp.maximum(obj, 0.0) - obj * fgf + jnp.log(1.0 + jnp.exp(-jnp.abs(obj)))
    p_t = prob * fgf + (1.0 - prob) * (1.0 - fgf)
    af = _FOCAL_ALPHA * fgf + (1.0 - _FOCAL_ALPHA) * (1.0 - fgf)
    one_mpt = 1.0 - p_t
    obj_loss = jnp.sum(ce_o * af * one_mpt * one_mpt) / float(P)

    ones_c = jnp.full((1, C), 1.0, jnp.float32)
    ones_g = jnp.full((1, G), 1.0, jnp.float32)

    masked = jnp.where(vcmc > 0.0, s, -10000.0)
    m_row = jnp.max(masked, axis=0, keepdims=True)
    ssum = jnp.dot(ones_c, jnp.exp(masked - m_row),
                   preferred_element_type=jnp.float32)
    lse = m_row + jnp.log(ssum)
    s_at = jnp.dot(ones_g, ohg * raw_cls,
                   preferred_element_type=jnp.float32)
    match_sum = jnp.sum((lse - s_at) * fgf)

    sig_at = jnp.dot(ones_g, ohg * cls_sig,
                     preferred_element_type=jnp.float32)
    pos_sum_b = jnp.sum(prob * sig_at * fgf)
    masked2 = jnp.where(vcmc > 0.0, s, -1e30)
    m2 = jnp.max(masked2, axis=0, keepdims=True)
    anyv = jnp.max(vcmc) > 0.0
    ncm = jnp.where(anyv, _sig(m2), 0.0)
    neg_sum_b = jnp.sum(prob * ncm * (1.0 - fgf))

    tx1 = tb[0:1, :]
    ty1 = tb[1:2, :]
    tx2 = tb[2:3, :]
    ty2 = tb[3:4, :]
    jx1 = jnp.maximum(bx1, tx1)
    jy1 = jnp.maximum(by1, ty1)
    jx2 = jnp.minimum(bx2, tx2)
    jy2 = jnp.minimum(by2, ty2)
    jint = jnp.maximum(jx2 - jx1, 0.0) * jnp.maximum(jy2 - jy1, 0.0)
    ta = jnp.maximum(tx2 - tx1, 0.0) * jnp.maximum(ty2 - ty1, 0.0)
    iou_r = jint / (pa + ta - jint + 1e-7)
    cw = jnp.maximum(bx2, tx2) - jnp.minimum(bx1, tx1)
    ch = jnp.maximum(by2, ty2) - jnp.minimum(by1, ty1)
    c2 = cw * cw + ch * ch + 1e-7
    rho2 = ((tx1 + tx2 - bx1 - bx2) ** 2 + (ty1 + ty2 - by1 - by2) ** 2) / 4.0
    w1 = jnp.maximum(bx2 - bx1, 1e-7)
    h1 = jnp.maximum(by2 - by1, 1e-7)
    w2 = jnp.maximum(tx2 - tx1, 1e-7)
    h2 = jnp.maximum(ty2 - ty1, 1e-7)
    dat = _atan_pos(w2 / h2) - _atan_pos(w1 / h1)
    v = (4.0 / (_PI * _PI)) * dat * dat
    alpha = v / (v - iou_r + (1.0 + 1e-7))
    ciou = iou_r - (rho2 / c2 + v * alpha)
    iou_sum = jnp.sum((1.0 - ciou) * fgf)

    x_full = d.astype(jnp.float32)
    dstack = jnp.concatenate([
        jnp.clip((ax - tx1) / st, 0.0, _REG_MAX - 1 - 0.01),
        jnp.clip((ay - ty1) / st, 0.0, _REG_MAX - 1 - 0.01),
        jnp.clip((tx2 - ax) / st, 0.0, _REG_MAX - 1 - 0.01),
        jnp.clip((ty2 - ay) / st, 0.0, _REG_MAX - 1 - 0.01),
    ], axis=0)
    grp = lax.broadcasted_iota(jnp.int32, (4, 4 * _REG_MAX), 1) // _REG_MAX
    smat = (grp == lax.broadcasted_iota(jnp.int32, (4, 4 * _REG_MAX), 0)
            ).astype(jnp.float32)
    dmatg = lax.broadcasted_iota(jnp.int32, (4 * _REG_MAX, 4), 0) // _REG_MAX
    dmat = (dmatg == lax.broadcasted_iota(jnp.int32, (4 * _REG_MAX, 4), 1)
            ).astype(jnp.float32)
    r16 = (lax.broadcasted_iota(jnp.int32, (4 * _REG_MAX, 1), 0)
           % _REG_MAX).astype(jnp.float32)
    mxg = jnp.concatenate(
        [jnp.max(x_full[k * _REG_MAX:(k + 1) * _REG_MAX, :], axis=0,
                 keepdims=True) for k in range(4)], axis=0)
    mxb = jnp.dot(dmat, mxg, preferred_element_type=jnp.float32)
    e = jnp.exp(x_full - mxb)
    sums = jnp.dot(smat, e, preferred_element_type=jnp.float32)
    lseg = mxg + jnp.log(sums)
    dful = jnp.dot(dmat, dstack, preferred_element_type=jnp.float32)
    w = jnp.maximum(1.0 - jnp.abs(r16 - dful), 0.0)
    wxs = jnp.dot(smat, w * x_full, preferred_element_type=jnp.float32)
    dfl_sum = jnp.sum((lseg - wxs) * fgf)

    has = cnt > 0.0
    acc_ref[0] = acc_ref[0] + obj_loss
    acc_ref[1] = acc_ref[1] + jnp.where(has, match_sum / jnp.maximum(cnt, 1.0), 0.0)
    acc_ref[2] = acc_ref[2] + jnp.where(has, iou_sum / jnp.maximum(cnt, 1.0), 0.0)
    acc_ref[3] = acc_ref[3] + jnp.where(has, dfl_sum / jnp.maximum(4.0 * cnt, 1.0), 0.0)
    acc_ref[4] = acc_ref[4] + pos_sum_b
    acc_ref[5] = acc_ref[5] + neg_sum_b
    acc_ref[6] = acc_ref[6] + cnt
    acc_ref[7] = acc_ref[7] + (float(P) - cnt)

    @pl.when(b == B - 1)
    def _fin():
        pos_mean = acc_ref[4] / jnp.maximum(acc_ref[6], 1.0)
        neg_mean = acc_ref[5] / jnp.maximum(acc_ref[7], 1.0)
        contrast = jnp.maximum(neg_mean - pos_mean + _MARGIN, 0.0)
        total = acc_ref[0] + acc_ref[1] + acc_ref[2] + acc_ref[3] + contrast
        lane = lax.broadcasted_iota(jnp.int32, (1, 8), 1)
        outv = (jnp.where(lane == 0, acc_ref[0], 0.0)
                + jnp.where(lane == 1, acc_ref[1], 0.0)
                + jnp.where(lane == 2, acc_ref[2], 0.0)
                + jnp.where(lane == 3, acc_ref[3], 0.0)
                + jnp.where(lane == 4, contrast, 0.0)
                + jnp.where(lane == 5, total, 0.0))
        out_ref[...] = outv


def kernel(pred_boxes, pred_scores, pred_objectness, anchor_points,
           stride_tensor, box_distribution, class_mask, gt_boxes, gt_labels):
    B, P, C = pred_scores.shape
    G = gt_boxes.shape[1]
    f32 = jnp.float32

    cls = gt_labels.astype(jnp.int32)
    cls_c = jnp.clip(cls, 0, C - 1)
    vcmc = class_mask.astype(f32).reshape(B, C, 1)
    valid = ((cls >= 0) & (cls < C)
             & jnp.take_along_axis(class_mask, cls_c, axis=1)).astype(f32)
    valid = valid.reshape(B, G, 1)
    onehotT = (cls_c[:, :, None]
               == jnp.arange(C, dtype=jnp.int32)[None, None, :]).astype(f32)
    gtb = gt_boxes.astype(f32)
    gtbT = jnp.transpose(gtb, (0, 2, 1))
    axr = jnp.broadcast_to(anchor_points[:, 0][None], (B, P))
    ayr = jnp.broadcast_to(anchor_points[:, 1][None], (B, P))
    str_ = jnp.broadcast_to(stride_tensor[None], (B, P))
    packT = jnp.stack(
        [pred_boxes[..., 0], pred_boxes[..., 1], pred_boxes[..., 2],
         pred_boxes[..., 3], pred_objectness, axr, ayr, str_],
        axis=1).astype(f32)
    scoresT = jnp.transpose(pred_scores.astype(f32), (0, 2, 1))
    distT = jnp.transpose(box_distribution.astype(f32), (0, 2, 1))

    body = functools.partial(_loss_body, B=B, P=P, C=C, G=G)
    out = pl.pallas_call(
        body,
        grid=(B,),
        in_specs=[
            pl.BlockSpec((1, G, 4), lambda b: (b, 0, 0)),
            pl.BlockSpec((1, 4, G), lambda b: (b, 0, 0)),
            pl.BlockSpec((1, G, C), lambda b: (b, 0, 0)),
            pl.BlockSpec((1, G, 1), lambda b: (b, 0, 0)),
            pl.BlockSpec((1, C, 1), lambda b: (b, 0, 0)),
            pl.BlockSpec((1, C, P), lambda b: (b, 0, 0)),
            pl.BlockSpec((1, 4 * _REG_MAX, P), lambda b: (b, 0, 0)),
            pl.BlockSpec((1, 8, P), lambda b: (b, 0, 0)),
        ],
        out_specs=pl.BlockSpec((1, 8), lambda b: (0, 0)),
        out_shape=jax.ShapeDtypeStruct((1, 8), f32),
        scratch_shapes=[pltpu.SMEM((8,), f32)],
    )(gtb, gtbT, onehotT, valid, vcmc, scoresT, distT, packT)
    return out[0, :6]
